# baseline (speedup 1.0000x reference)
import functools

import jax
import jax.numpy as jnp
from jax import lax
from jax.experimental import pallas as pl
from jax.experimental.pallas import tpu as pltpu
from jax.experimental.pallas import tpu_sc as plsc

E = 16
TOPK = 2
H = 2048
I = 1536
T = 128
T2 = 256
_NEG = -1e30


def _router_body(x_ref, gwp_ref, idx_ref, w_ref):
    xb = x_ref[...]
    logits = lax.dot_general(xb, gwp_ref[...], (((1,), (1,)), ((), ())),
                             preferred_element_type=jnp.float32)
    ids = lax.broadcasted_iota(jnp.int32, logits.shape, 1)
    valid = ids < E
    l1 = jnp.where(valid, logits, _NEG)
    m1 = jnp.max(l1, axis=1, keepdims=True)
    big = jnp.int32(2**30)
    i1 = jnp.min(jnp.where(l1 == m1, ids, big), axis=1, keepdims=True)
    l2 = jnp.where(ids == i1, _NEG, l1)
    m2 = jnp.max(l2, axis=1, keepdims=True)
    i2 = jnp.min(jnp.where(l2 == m2, ids, big), axis=1, keepdims=True)
    ex = jnp.exp(m2 - m1)
    w1 = 1.0 / (1.0 + ex)
    w2 = ex / (1.0 + ex)
    col0 = ids == 0
    idx_ref[...] = jnp.where(col0, i1, i2)
    w_ref[...] = jnp.where(col0, w1, w2)


def _router(xf, gwp):
    nt = xf.shape[0]
    grid = (nt // T2,)
    return pl.pallas_call(
        _router_body,
        grid=grid,
        in_specs=[pl.BlockSpec((T2, H), lambda r: (r, 0)),
                  pl.BlockSpec((128, H), lambda r: (0, 0))],
        out_specs=[pl.BlockSpec((T2, 128), lambda r: (r, 0)),
                   pl.BlockSpec((T2, 128), lambda r: (r, 0))],
        out_shape=[jax.ShapeDtypeStruct((nt, 128), jnp.int32),
                   jax.ShapeDtypeStruct((nt, 128), jnp.float32)],
    )(xf, gwp)


def _sc_gather(table, idx):
    B = idx.shape[0]
    D = table.shape[1]
    NC, NS = 2, 16
    NW = NC * NS
    chunk = 16
    bpw = B // NW
    nch = bpw // chunk
    mesh = plsc.VectorSubcoreMesh(core_axis_name="c", subcore_axis_name="s",
                                  num_cores=NC, num_subcores=NS)

    @functools.partial(
        pl.kernel, mesh=mesh,
        out_type=jax.ShapeDtypeStruct((B, D), table.dtype),
        scratch_types=[
            pltpu.VMEM((nch, chunk), jnp.int32),
            pltpu.VMEM((chunk, D), table.dtype),
            pltpu.VMEM((chunk, D), table.dtype),
            pltpu.SemaphoreType.DMA,
            pltpu.SemaphoreType.DMA,
        ],
    )
    def k(table_hbm, idx_hbm, out_hbm, idx_v, rows_a, rows_b, sem_a, sem_b):
        wid = lax.axis_index("s") * NC + lax.axis_index("c")
        base = wid * bpw
        pltpu.sync_copy(idx_hbm.at[wid], idx_v)

        def start(j, rows, sem):
            pltpu.async_copy(table_hbm.at[idx_v.at[j]], rows, sem)

        def wait(j, rows, sem):
            pltpu.make_async_copy(table_hbm.at[idx_v.at[j]], rows, sem).wait()

        start(0, rows_a, sem_a)

        def body(jj, carry):
            j = jj * 2
            start(j + 1, rows_b, sem_b)
            wait(j, rows_a, sem_a)
            pltpu.sync_copy(rows_a, out_hbm.at[pl.ds(base + j * chunk, chunk)])

            @pl.when(j + 2 < nch)
            def _():
                start(j + 2, rows_a, sem_a)

            wait(j + 1, rows_b, sem_b)
            pltpu.sync_copy(rows_b,
                            out_hbm.at[pl.ds(base + (j + 1) * chunk, chunk)])
            return carry

        lax.fori_loop(0, nch // 2, body, 0)

    return k(table, idx.reshape(NW, nch, chunk))


def _sc_combine(outs, pos_a, pos_b, sh):
    B = pos_a.shape[0]
    D = outs.shape[1]
    NC, NS = 2, 16
    NW = NC * NS
    chunk = 8
    bpw = B // NW
    nch = bpw // chunk
    mesh = plsc.VectorSubcoreMesh(core_axis_name="c", subcore_axis_name="s",
                                  num_cores=NC, num_subcores=NS)

    buf = lambda: pltpu.VMEM((chunk, D), jnp.float32)

    @functools.partial(
        pl.kernel, mesh=mesh,
        out_type=jax.ShapeDtypeStruct((B, D), jnp.float32),
        scratch_types=[
            pltpu.VMEM((nch, chunk), jnp.int32),
            pltpu.VMEM((nch, chunk), jnp.int32),
            [buf(), buf(), buf()],
            [buf(), buf(), buf()],
            [pltpu.SemaphoreType.DMA] * 3,
            [pltpu.SemaphoreType.DMA] * 3,
        ],
    )
    def k(outs_hbm, pa_hbm, pb_hbm, sh_hbm, comb_hbm,
          pa_v, pb_v, bufs0, bufs1, sems0, sems1):
        wid = lax.axis_index("s") * NC + lax.axis_index("c")
        base = wid * bpw
        pltpu.sync_copy(pa_hbm.at[wid], pa_v)
        pltpu.sync_copy(pb_hbm.at[wid], pb_v)
        rings = (list(zip(bufs0, sems0)), list(zip(bufs1, sems1)))

        def start(j, ring):
            (va, sa), (vb, sb), (vs, ss) = ring
            pltpu.async_copy(outs_hbm.at[pa_v.at[j]], va, sa)
            pltpu.async_copy(outs_hbm.at[pb_v.at[j]], vb, sb)
            pltpu.async_copy(sh_hbm.at[pl.ds(base + j * chunk, chunk)], vs, ss)

        def finish(j, ring):
            (va, sa), (vb, sb), (vs, ss) = ring
            pltpu.make_async_copy(outs_hbm.at[pa_v.at[j]], va, sa).wait()
            pltpu.make_async_copy(outs_hbm.at[pb_v.at[j]], vb, sb).wait()
            pltpu.make_async_copy(
                sh_hbm.at[pl.ds(base + j * chunk, chunk)], vs, ss).wait()

            def add_body(c, carry2):
                sl = pl.ds(c * 16, 16)
                for r in range(chunk):
                    va[r, sl] = va[r, sl] + vb[r, sl] + vs[r, sl]
                return carry2

            lax.fori_loop(0, D // 16, add_body, 0)
            pltpu.sync_copy(va, comb_hbm.at[pl.ds(base + j * chunk, chunk)])

        start(0, rings[0])

        def body(jj, carry):
            j = jj * 2
            start(j + 1, rings[1])
            finish(j, rings[0])

            @pl.when(j + 2 < nch)
            def _():
                start(j + 2, rings[0])

            finish(j + 1, rings[1])
            return carry

        lax.fori_loop(0, nch // 2, body, 0)

    return k(outs, pos_a.reshape(NW, nch, chunk), pos_b.reshape(NW, nch, chunk),
             sh)


def _k1_body(te_ref, xs_ref, gw_ref, uw_ref, wv_ref, h_ref):
    xb = xs_ref[...].astype(jnp.float32)
    g = lax.dot_general(xb, gw_ref[0], (((1,), (1,)), ((), ())),
                        preferred_element_type=jnp.float32)
    u = lax.dot_general(xb, uw_ref[0], (((1,), (1,)), ((), ())),
                        preferred_element_type=jnp.float32)
    h_ref[...] = ((g * jax.nn.sigmoid(g)) * u * wv_ref[...]).astype(h_ref.dtype)


def _k1(xs, gw, uw, wv, tile_e):
    n_pad = xs.shape[0]
    n_r = n_pad // T
    gs = pltpu.PrefetchScalarGridSpec(
        num_scalar_prefetch=1,
        grid=(n_r,),
        in_specs=[pl.BlockSpec((T, H), lambda r, te: (r, 0)),
                  pl.BlockSpec((1, I, H), lambda r, te: (te[r], 0, 0)),
                  pl.BlockSpec((1, I, H), lambda r, te: (te[r], 0, 0)),
                  pl.BlockSpec((T, 1), lambda r, te: (r, 0))],
        out_specs=pl.BlockSpec((T, I), lambda r, te: (r, 0)),
    )
    return pl.pallas_call(
        _k1_body, grid_spec=gs,
        out_shape=jax.ShapeDtypeStruct((n_pad, I), jnp.bfloat16),
    )(tile_e, xs, gw, uw, wv)


def _k2_body(te_ref, h_ref, dw_ref, o_ref):
    hb = h_ref[...].astype(jnp.float32)
    o_ref[...] = lax.dot_general(hb, dw_ref[0], (((1,), (1,)), ((), ())),
                                 preferred_element_type=jnp.float32)


def _k2(h, dw, tile_e):
    n_pad = h.shape[0]
    n_r = n_pad // T
    gs = pltpu.PrefetchScalarGridSpec(
        num_scalar_prefetch=1,
        grid=(n_r,),
        in_specs=[pl.BlockSpec((T, I), lambda r, te: (r, 0)),
                  pl.BlockSpec((1, H, I), lambda r, te: (te[r], 0, 0))],
        out_specs=pl.BlockSpec((T, H), lambda r, te: (r, 0)),
    )
    return pl.pallas_call(
        _k2_body, grid_spec=gs,
        out_shape=jax.ShapeDtypeStruct((n_pad, H), jnp.float32),
    )(tile_e, h, dw)


def _shared_body(x_ref, gw_ref, uw_ref, dw_ref, o_ref):
    xb = x_ref[...]
    g = lax.dot_general(xb, gw_ref[...], (((1,), (1,)), ((), ())),
                        preferred_element_type=jnp.float32)
    u = lax.dot_general(xb, uw_ref[...], (((1,), (1,)), ((), ())),
                        preferred_element_type=jnp.float32)
    h = (g * jax.nn.sigmoid(g)) * u
    o_ref[...] = lax.dot_general(h, dw_ref[...], (((1,), (1,)), ((), ())),
                                 preferred_element_type=jnp.float32)


def _shared(xf, sgw, suw, sdw):
    nt = xf.shape[0]
    return pl.pallas_call(
        _shared_body,
        grid=(nt // T2,),
        in_specs=[pl.BlockSpec((T2, H), lambda r: (r, 0)),
                  pl.BlockSpec((I, H), lambda r: (0, 0)),
                  pl.BlockSpec((I, H), lambda r: (0, 0)),
                  pl.BlockSpec((H, I), lambda r: (0, 0))],
        out_specs=pl.BlockSpec((T2, H), lambda r: (r, 0)),
        out_shape=jax.ShapeDtypeStruct((nt, H), jnp.float32),
    )(xf, sgw, suw, sdw)


def _add_body(a_ref, b_ref, c_ref, o_ref):
    o_ref[...] = a_ref[...] + b_ref[...] + c_ref[...]


def _add3(a, b, c):
    nt = a.shape[0]
    spec = pl.BlockSpec((T2, H), lambda r: (r, 0))
    return pl.pallas_call(
        _add_body,
        grid=(nt // T2,),
        in_specs=[spec, spec, spec],
        out_specs=spec,
        out_shape=jax.ShapeDtypeStruct((nt, H), jnp.float32),
    )(a, b, c)


def kernel(x, gate_w, expert_gate_w, expert_up_w, expert_down_w,
           shared_gate_w, shared_up_w, shared_down_w):
    orig_shape = x.shape
    xf = x.reshape(-1, H)
    nt = xf.shape[0]
    n_pad = nt * TOPK + E * T
    n_r_tiles = n_pad // T

    gwp = jnp.zeros((128, H), jnp.float32).at[:E].set(gate_w)
    idx_full, w_full = _router(xf, gwp)
    ti = idx_full[:, :TOPK]
    tw = w_full[:, :TOPK]

    flat_e = ti.reshape(-1)
    oh = (flat_e[:, None] == jnp.arange(E, dtype=flat_e.dtype)).astype(jnp.int32)
    counts = oh.sum(axis=0)
    padded = ((counts + T - 1) // T) * T
    offs = jnp.concatenate([jnp.zeros((1,), padded.dtype),
                            jnp.cumsum(padded)[:-1]])
    ranks = jnp.sum((jnp.cumsum(oh, axis=0) - oh) * oh, axis=1)
    dest = (jnp.sum(oh * offs[None, :], axis=1) + ranks).astype(jnp.int32)
    tok_ids = jnp.arange(nt * TOPK, dtype=jnp.int32) // TOPK
    pairs = jnp.stack([tok_ids.astype(jnp.float32), tw.reshape(-1)], axis=1)
    slots = jnp.zeros((n_pad, 2), jnp.float32).at[dest].set(pairs)
    src_tok = slots[:, 0].astype(jnp.int32)
    wv = slots[:, 1:2]
    tile_start = jnp.arange(n_r_tiles, dtype=offs.dtype)[:, None] * T
    tile_e = (jnp.sum((tile_start >= offs[None, :]).astype(jnp.int32), axis=1)
              - 1).clip(0, E - 1).astype(jnp.int32)

    sh = _shared(xf, shared_gate_w, shared_up_w, shared_down_w)
    xi = lax.bitcast_convert_type(
        xf.astype(jnp.bfloat16).reshape(nt, H // 2, 2), jnp.int32)
    xs = lax.bitcast_convert_type(
        _sc_gather(xi, src_tok), jnp.bfloat16).reshape(n_pad, H)
    h = _k1(xs, expert_gate_w, expert_up_w, wv, tile_e)
    outs = _k2(h, expert_down_w, tile_e)

    pos = dest.reshape(nt, TOPK)
    comb = _sc_combine(outs, pos[:, 0], pos[:, 1], sh)
    return comb.reshape(orig_shape)


# speedup vs baseline: 1.5601x; 1.5601x over previous
import functools

import jax
import jax.numpy as jnp
from jax import lax
from jax.experimental import pallas as pl
from jax.experimental.pallas import tpu as pltpu
from jax.experimental.pallas import tpu_sc as plsc

E = 16
TOPK = 2
H = 2048
I = 1536
T = 128
T2 = 256
_NEG = -1e30


def _router_body(x_ref, gwp_ref, idx_ref, w_ref):
    xb = x_ref[...]
    logits = lax.dot_general(xb, gwp_ref[...], (((1,), (1,)), ((), ())),
                             preferred_element_type=jnp.float32)
    ids = lax.broadcasted_iota(jnp.int32, logits.shape, 1)
    valid = ids < E
    l1 = jnp.where(valid, logits, _NEG)
    m1 = jnp.max(l1, axis=1, keepdims=True)
    big = jnp.int32(2**30)
    i1 = jnp.min(jnp.where(l1 == m1, ids, big), axis=1, keepdims=True)
    l2 = jnp.where(ids == i1, _NEG, l1)
    m2 = jnp.max(l2, axis=1, keepdims=True)
    i2 = jnp.min(jnp.where(l2 == m2, ids, big), axis=1, keepdims=True)
    ex = jnp.exp(m2 - m1)
    w1 = 1.0 / (1.0 + ex)
    w2 = ex / (1.0 + ex)
    col0 = ids == 0
    idx_ref[...] = jnp.where(col0, i1, i2)
    w_ref[...] = jnp.where(col0, w1, w2)


def _router(xf, gwp):
    nt = xf.shape[0]
    grid = (nt // T2,)
    return pl.pallas_call(
        _router_body,
        grid=grid,
        in_specs=[pl.BlockSpec((T2, H), lambda r: (r, 0)),
                  pl.BlockSpec((128, H), lambda r: (0, 0))],
        out_specs=[pl.BlockSpec((T2, 128), lambda r: (r, 0)),
                   pl.BlockSpec((T2, 128), lambda r: (r, 0))],
        out_shape=[jax.ShapeDtypeStruct((nt, 128), jnp.int32),
                   jax.ShapeDtypeStruct((nt, 128), jnp.float32)],
    )(xf, gwp)


def _sc_gather(table, idx):
    B = idx.shape[0]
    D = table.shape[1]
    NC, NS = 2, 16
    NW = NC * NS
    chunk = 16
    bpw = B // NW
    nch = bpw // chunk
    mesh = plsc.VectorSubcoreMesh(core_axis_name="c", subcore_axis_name="s",
                                  num_cores=NC, num_subcores=NS)

    @functools.partial(
        pl.kernel, mesh=mesh,
        out_type=jax.ShapeDtypeStruct((B, D), table.dtype),
        scratch_types=[
            pltpu.VMEM((nch, chunk), jnp.int32),
            pltpu.VMEM((chunk, D), table.dtype),
            pltpu.VMEM((chunk, D), table.dtype),
            pltpu.SemaphoreType.DMA,
            pltpu.SemaphoreType.DMA,
        ],
    )
    def k(table_hbm, idx_hbm, out_hbm, idx_v, rows_a, rows_b, sem_a, sem_b):
        wid = lax.axis_index("s") * NC + lax.axis_index("c")
        base = wid * bpw
        pltpu.sync_copy(idx_hbm.at[wid], idx_v)

        def start(j, rows, sem):
            pltpu.async_copy(table_hbm.at[idx_v.at[j]], rows, sem)

        def wait(j, rows, sem):
            pltpu.make_async_copy(table_hbm.at[idx_v.at[j]], rows, sem).wait()

        start(0, rows_a, sem_a)

        def body(jj, carry):
            j = jj * 2
            start(j + 1, rows_b, sem_b)
            wait(j, rows_a, sem_a)
            pltpu.sync_copy(rows_a, out_hbm.at[pl.ds(base + j * chunk, chunk)])

            @pl.when(j + 2 < nch)
            def _():
                start(j + 2, rows_a, sem_a)

            wait(j + 1, rows_b, sem_b)
            pltpu.sync_copy(rows_b,
                            out_hbm.at[pl.ds(base + (j + 1) * chunk, chunk)])
            return carry

        lax.fori_loop(0, nch // 2, body, 0)

    return k(table, idx.reshape(NW, nch, chunk))


def _sc_combine(outs, pos_a, pos_b, sh):
    B = pos_a.shape[0]
    D = outs.shape[1]
    NC, NS = 2, 16
    NW = NC * NS
    chunk = 8
    bpw = B // NW
    nch = bpw // chunk
    mesh = plsc.VectorSubcoreMesh(core_axis_name="c", subcore_axis_name="s",
                                  num_cores=NC, num_subcores=NS)

    buf = lambda: pltpu.VMEM((chunk, D), jnp.float32)

    @functools.partial(
        pl.kernel, mesh=mesh,
        out_type=jax.ShapeDtypeStruct((B, D), jnp.float32),
        scratch_types=[
            pltpu.VMEM((nch, chunk), jnp.int32),
            pltpu.VMEM((nch, chunk), jnp.int32),
            [buf(), buf(), buf()],
            [buf(), buf(), buf()],
            [pltpu.SemaphoreType.DMA] * 3,
            [pltpu.SemaphoreType.DMA] * 3,
        ],
    )
    def k(outs_hbm, pa_hbm, pb_hbm, sh_hbm, comb_hbm,
          pa_v, pb_v, bufs0, bufs1, sems0, sems1):
        wid = lax.axis_index("s") * NC + lax.axis_index("c")
        base = wid * bpw
        pltpu.sync_copy(pa_hbm.at[wid], pa_v)
        pltpu.sync_copy(pb_hbm.at[wid], pb_v)
        rings = (list(zip(bufs0, sems0)), list(zip(bufs1, sems1)))

        def start(j, ring):
            (va, sa), (vb, sb), (vs, ss) = ring
            pltpu.async_copy(outs_hbm.at[pa_v.at[j]], va, sa)
            pltpu.async_copy(outs_hbm.at[pb_v.at[j]], vb, sb)
            pltpu.async_copy(sh_hbm.at[pl.ds(base + j * chunk, chunk)], vs, ss)

        def finish(j, ring):
            (va, sa), (vb, sb), (vs, ss) = ring
            pltpu.make_async_copy(outs_hbm.at[pa_v.at[j]], va, sa).wait()
            pltpu.make_async_copy(outs_hbm.at[pb_v.at[j]], vb, sb).wait()
            pltpu.make_async_copy(
                sh_hbm.at[pl.ds(base + j * chunk, chunk)], vs, ss).wait()

            def add_body(c, carry2):
                sl = pl.ds(c * 16, 16)
                for r in range(chunk):
                    va[r, sl] = va[r, sl] + vb[r, sl] + vs[r, sl]
                return carry2

            lax.fori_loop(0, D // 16, add_body, 0)
            pltpu.sync_copy(va, comb_hbm.at[pl.ds(base + j * chunk, chunk)])

        start(0, rings[0])

        def body(jj, carry):
            j = jj * 2
            start(j + 1, rings[1])
            finish(j, rings[0])

            @pl.when(j + 2 < nch)
            def _():
                start(j + 2, rings[0])

            finish(j + 1, rings[1])
            return carry

        lax.fori_loop(0, nch // 2, body, 0)

    return k(outs, pos_a.reshape(NW, nch, chunk), pos_b.reshape(NW, nch, chunk),
             sh)


def _k1_body(te_ref, xs_ref, gw_ref, uw_ref, wv_ref, h_ref):
    xb = xs_ref[...]
    g = lax.dot_general(xb, gw_ref[0], (((1,), (1,)), ((), ())),
                        preferred_element_type=jnp.float32)
    u = lax.dot_general(xb, uw_ref[0], (((1,), (1,)), ((), ())),
                        preferred_element_type=jnp.float32)
    h_ref[...] = ((g * jax.nn.sigmoid(g)) * u * wv_ref[...]).astype(h_ref.dtype)


def _k1(xs, gw, uw, wv, tile_e):
    n_pad = xs.shape[0]
    n_r = n_pad // T
    gs = pltpu.PrefetchScalarGridSpec(
        num_scalar_prefetch=1,
        grid=(n_r,),
        in_specs=[pl.BlockSpec((T, H), lambda r, te: (r, 0)),
                  pl.BlockSpec((1, I, H), lambda r, te: (te[r], 0, 0)),
                  pl.BlockSpec((1, I, H), lambda r, te: (te[r], 0, 0)),
                  pl.BlockSpec((T, 1), lambda r, te: (r, 0))],
        out_specs=pl.BlockSpec((T, I), lambda r, te: (r, 0)),
    )
    return pl.pallas_call(
        _k1_body, grid_spec=gs,
        out_shape=jax.ShapeDtypeStruct((n_pad, I), jnp.bfloat16),
    )(tile_e, xs, gw, uw, wv)


def _k2_body(te_ref, h_ref, dw_ref, o_ref):
    hb = h_ref[...].astype(jnp.float32)
    o_ref[...] = lax.dot_general(hb, dw_ref[0], (((1,), (1,)), ((), ())),
                                 preferred_element_type=jnp.float32)


def _k2(h, dw, tile_e):
    n_pad = h.shape[0]
    n_r = n_pad // T
    gs = pltpu.PrefetchScalarGridSpec(
        num_scalar_prefetch=1,
        grid=(n_r,),
        in_specs=[pl.BlockSpec((T, I), lambda r, te: (r, 0)),
                  pl.BlockSpec((1, H, I), lambda r, te: (te[r], 0, 0))],
        out_specs=pl.BlockSpec((T, H), lambda r, te: (r, 0)),
    )
    return pl.pallas_call(
        _k2_body, grid_spec=gs,
        out_shape=jax.ShapeDtypeStruct((n_pad, H), jnp.float32),
    )(tile_e, h, dw)


def _shared_body(x_ref, gw_ref, uw_ref, dw_ref, o_ref):
    xb = x_ref[...]
    g = lax.dot_general(xb, gw_ref[...], (((1,), (1,)), ((), ())),
                        preferred_element_type=jnp.float32)
    u = lax.dot_general(xb, uw_ref[...], (((1,), (1,)), ((), ())),
                        preferred_element_type=jnp.float32)
    h = (g * jax.nn.sigmoid(g)) * u
    o_ref[...] = lax.dot_general(h, dw_ref[...], (((1,), (1,)), ((), ())),
                                 preferred_element_type=jnp.float32)


def _shared(xf, sgw, suw, sdw):
    nt = xf.shape[0]
    return pl.pallas_call(
        _shared_body,
        grid=(nt // T2,),
        in_specs=[pl.BlockSpec((T2, H), lambda r: (r, 0)),
                  pl.BlockSpec((I, H), lambda r: (0, 0)),
                  pl.BlockSpec((I, H), lambda r: (0, 0)),
                  pl.BlockSpec((H, I), lambda r: (0, 0))],
        out_specs=pl.BlockSpec((T2, H), lambda r: (r, 0)),
        out_shape=jax.ShapeDtypeStruct((nt, H), jnp.float32),
    )(xf, sgw, suw, sdw)


def _add_body(a_ref, b_ref, c_ref, o_ref):
    o_ref[...] = a_ref[...] + b_ref[...] + c_ref[...]


def _add3(a, b, c):
    nt = a.shape[0]
    spec = pl.BlockSpec((T2, H), lambda r: (r, 0))
    return pl.pallas_call(
        _add_body,
        grid=(nt // T2,),
        in_specs=[spec, spec, spec],
        out_specs=spec,
        out_shape=jax.ShapeDtypeStruct((nt, H), jnp.float32),
    )(a, b, c)


def kernel(x, gate_w, expert_gate_w, expert_up_w, expert_down_w,
           shared_gate_w, shared_up_w, shared_down_w):
    orig_shape = x.shape
    xf = x.reshape(-1, H)
    nt = xf.shape[0]
    n_pad = nt * TOPK + E * T
    n_r_tiles = n_pad // T

    gwp = jnp.zeros((128, H), jnp.float32).at[:E].set(gate_w)
    idx_full, w_full = _router(xf, gwp)
    ti = idx_full[:, :TOPK]
    tw = w_full[:, :TOPK]

    flat_e = ti.reshape(-1)
    oh = (flat_e[:, None] == jnp.arange(E, dtype=flat_e.dtype)).astype(jnp.int32)
    counts = oh.sum(axis=0)
    padded = ((counts + T - 1) // T) * T
    offs = jnp.concatenate([jnp.zeros((1,), padded.dtype),
                            jnp.cumsum(padded)[:-1]])
    ranks = jnp.sum((jnp.cumsum(oh, axis=0) - oh) * oh, axis=1)
    dest = (jnp.sum(oh * offs[None, :], axis=1) + ranks).astype(jnp.int32)
    tok_ids = jnp.arange(nt * TOPK, dtype=jnp.int32) // TOPK
    pairs = jnp.stack([tok_ids.astype(jnp.float32), tw.reshape(-1)], axis=1)
    slots = jnp.zeros((n_pad, 2), jnp.float32).at[dest].set(pairs)
    src_tok = slots[:, 0].astype(jnp.int32)
    wv = slots[:, 1:2]
    tile_start = jnp.arange(n_r_tiles, dtype=offs.dtype)[:, None] * T
    tile_e = (jnp.sum((tile_start >= offs[None, :]).astype(jnp.int32), axis=1)
              - 1).clip(0, E - 1).astype(jnp.int32)

    sh = _shared(xf, shared_gate_w, shared_up_w, shared_down_w)
    xs = _sc_gather(xf, src_tok)
    h = _k1(xs, expert_gate_w, expert_up_w, wv, tile_e)
    outs = _k2(h, expert_down_w, tile_e)

    pos = dest.reshape(nt, TOPK)
    comb = _sc_combine(outs, pos[:, 0], pos[:, 1], sh)
    return comb.reshape(orig_shape)


# speedup vs baseline: 1.7029x; 1.0915x over previous
import functools

import jax
import jax.numpy as jnp
from jax import lax
from jax.experimental import pallas as pl
from jax.experimental.pallas import tpu as pltpu
from jax.experimental.pallas import tpu_sc as plsc

E = 16
TOPK = 2
H = 2048
I = 1536
T = 128
T2 = 256
_NEG = -1e30


def _router_body(x_ref, gwp_ref, idx_ref, w_ref):
    xb = x_ref[...]
    logits = lax.dot_general(xb, gwp_ref[...], (((1,), (1,)), ((), ())),
                             preferred_element_type=jnp.float32)
    ids = lax.broadcasted_iota(jnp.int32, logits.shape, 1)
    valid = ids < E
    l1 = jnp.where(valid, logits, _NEG)
    m1 = jnp.max(l1, axis=1, keepdims=True)
    big = jnp.int32(2**30)
    i1 = jnp.min(jnp.where(l1 == m1, ids, big), axis=1, keepdims=True)
    l2 = jnp.where(ids == i1, _NEG, l1)
    m2 = jnp.max(l2, axis=1, keepdims=True)
    i2 = jnp.min(jnp.where(l2 == m2, ids, big), axis=1, keepdims=True)
    ex = jnp.exp(m2 - m1)
    w1 = 1.0 / (1.0 + ex)
    w2 = ex / (1.0 + ex)
    col0 = ids == 0
    idx_ref[...] = jnp.where(col0, i1, i2)
    w_ref[...] = jnp.where(col0, w1, w2)


def _router(xf, gwp):
    nt = xf.shape[0]
    grid = (nt // T2,)
    return pl.pallas_call(
        _router_body,
        grid=grid,
        in_specs=[pl.BlockSpec((T2, H), lambda r: (r, 0)),
                  pl.BlockSpec((128, H), lambda r: (0, 0))],
        out_specs=[pl.BlockSpec((T2, 128), lambda r: (r, 0)),
                   pl.BlockSpec((T2, 128), lambda r: (r, 0))],
        out_shape=[jax.ShapeDtypeStruct((nt, 128), jnp.int32),
                   jax.ShapeDtypeStruct((nt, 128), jnp.float32)],
    )(xf, gwp)


def _sc_gather(table, idx):
    B = idx.shape[0]
    D = table.shape[1]
    NC, NS = 2, 16
    NW = NC * NS
    chunk = 16
    bpw = B // NW
    nch = bpw // chunk
    mesh = plsc.VectorSubcoreMesh(core_axis_name="c", subcore_axis_name="s",
                                  num_cores=NC, num_subcores=NS)

    @functools.partial(
        pl.kernel, mesh=mesh,
        out_type=jax.ShapeDtypeStruct((B, D), table.dtype),
        scratch_types=[
            pltpu.VMEM((nch, chunk), jnp.int32),
            pltpu.VMEM((chunk, D), table.dtype),
            pltpu.VMEM((chunk, D), table.dtype),
            pltpu.SemaphoreType.DMA,
            pltpu.SemaphoreType.DMA,
        ],
    )
    def k(table_hbm, idx_hbm, out_hbm, idx_v, rows_a, rows_b, sem_a, sem_b):
        wid = lax.axis_index("s") * NC + lax.axis_index("c")
        base = wid * bpw
        pltpu.sync_copy(idx_hbm.at[wid], idx_v)

        def start(j, rows, sem):
            pltpu.async_copy(table_hbm.at[idx_v.at[j]], rows, sem)

        def wait(j, rows, sem):
            pltpu.make_async_copy(table_hbm.at[idx_v.at[j]], rows, sem).wait()

        start(0, rows_a, sem_a)

        def body(jj, carry):
            j = jj * 2
            start(j + 1, rows_b, sem_b)
            wait(j, rows_a, sem_a)
            pltpu.sync_copy(rows_a, out_hbm.at[pl.ds(base + j * chunk, chunk)])

            @pl.when(j + 2 < nch)
            def _():
                start(j + 2, rows_a, sem_a)

            wait(j + 1, rows_b, sem_b)
            pltpu.sync_copy(rows_b,
                            out_hbm.at[pl.ds(base + (j + 1) * chunk, chunk)])
            return carry

        lax.fori_loop(0, nch // 2, body, 0)

    return k(table, idx.reshape(NW, nch, chunk))


def _sc_combine(outs, pos_a, pos_b, sh):
    B = pos_a.shape[0]
    D = outs.shape[1]
    NC, NS = 2, 16
    NW = NC * NS
    chunk = 8
    bpw = B // NW
    nch = bpw // chunk
    mesh = plsc.VectorSubcoreMesh(core_axis_name="c", subcore_axis_name="s",
                                  num_cores=NC, num_subcores=NS)

    buf = lambda: pltpu.VMEM((chunk, D), jnp.float32)

    @functools.partial(
        pl.kernel, mesh=mesh,
        out_type=jax.ShapeDtypeStruct((B, D), jnp.float32),
        scratch_types=[
            pltpu.VMEM((nch, chunk), jnp.int32),
            pltpu.VMEM((nch, chunk), jnp.int32),
            [buf(), buf(), buf()],
            [buf(), buf(), buf()],
            [pltpu.SemaphoreType.DMA] * 3,
            [pltpu.SemaphoreType.DMA] * 3,
        ],
    )
    def k(outs_hbm, pa_hbm, pb_hbm, sh_hbm, comb_hbm,
          pa_v, pb_v, bufs0, bufs1, sems0, sems1):
        wid = lax.axis_index("s") * NC + lax.axis_index("c")
        base = wid * bpw
        pltpu.sync_copy(pa_hbm.at[wid], pa_v)
        pltpu.sync_copy(pb_hbm.at[wid], pb_v)
        rings = (list(zip(bufs0, sems0)), list(zip(bufs1, sems1)))

        def start(j, ring):
            (va, sa), (vb, sb), (vs, ss) = ring
            pltpu.async_copy(outs_hbm.at[pa_v.at[j]], va, sa)
            pltpu.async_copy(outs_hbm.at[pb_v.at[j]], vb, sb)
            pltpu.async_copy(sh_hbm.at[pl.ds(base + j * chunk, chunk)], vs, ss)

        def finish(j, ring):
            (va, sa), (vb, sb), (vs, ss) = ring
            pltpu.make_async_copy(outs_hbm.at[pa_v.at[j]], va, sa).wait()
            pltpu.make_async_copy(outs_hbm.at[pb_v.at[j]], vb, sb).wait()
            pltpu.make_async_copy(
                sh_hbm.at[pl.ds(base + j * chunk, chunk)], vs, ss).wait()

            def add_body(c, carry2):
                sl = pl.ds(c * 16, 16)
                for r in range(chunk):
                    va[r, sl] = va[r, sl] + vb[r, sl] + vs[r, sl]
                return carry2

            lax.fori_loop(0, D // 16, add_body, 0)
            pltpu.sync_copy(va, comb_hbm.at[pl.ds(base + j * chunk, chunk)])

        start(0, rings[0])

        def body(jj, carry):
            j = jj * 2
            start(j + 1, rings[1])
            finish(j, rings[0])

            @pl.when(j + 2 < nch)
            def _():
                start(j + 2, rings[0])

            finish(j + 1, rings[1])
            return carry

        lax.fori_loop(0, nch // 2, body, 0)

    return k(outs, pos_a.reshape(NW, nch, chunk), pos_b.reshape(NW, nch, chunk),
             sh)


def _weight_sched(tile_e, padded):
    n_r = tile_e.shape[0]
    ee = jnp.arange(E, dtype=tile_e.dtype)
    change = jnp.concatenate([jnp.ones((1,), jnp.bool_),
                              tile_e[1:] != tile_e[:-1]])
    run_id = jnp.cumsum(change.astype(jnp.int32)) - 1
    slot = run_id % 2
    present = padded > 0
    cand = jnp.where((ee[None, :] > ee[:, None]) & present[None, :],
                     ee[None, :], E)
    nxt_tab = jnp.min(cand, axis=1)
    oh_t = (tile_e[:, None] == ee[None, :]).astype(jnp.int32)
    nxt = jnp.sum(oh_t * nxt_tab[None, :], axis=1)
    return jnp.stack([tile_e.astype(jnp.int32), change.astype(jnp.int32),
                      slot.astype(jnp.int32), nxt.astype(jnp.int32)])


def _run_prefetch(scal_ref, r, w_hbms, bufs0, bufs1, sems0, sems1):
    e = scal_ref[0, r]
    fr = scal_ref[1, r]
    slot = scal_ref[2, r]
    nxt = scal_ref[3, r]

    def issue(eid, to_slot):
        @pl.when(to_slot == 0)
        def _():
            for w, b, s in zip(w_hbms, bufs0, sems0):
                pltpu.async_copy(w.at[eid], b, s)

        @pl.when(to_slot == 1)
        def _():
            for w, b, s in zip(w_hbms, bufs1, sems1):
                pltpu.async_copy(w.at[eid], b, s)

    @pl.when(r == 0)
    def _():
        issue(e, slot)

    @pl.when(fr == 1)
    def _():
        @pl.when(slot == 0)
        def _():
            for w, b, s in zip(w_hbms, bufs0, sems0):
                pltpu.make_async_copy(w.at[e], b, s).wait()

        @pl.when(slot == 1)
        def _():
            for w, b, s in zip(w_hbms, bufs1, sems1):
                pltpu.make_async_copy(w.at[e], b, s).wait()

        @pl.when(nxt < E)
        def _():
            issue(nxt, 1 - slot)

    return slot


def _k1_body(scal_ref, xs_ref, gw_hbm, uw_hbm, wv_ref, h_ref,
             g0, g1, u0, u1, sg0, sg1, su0, su1):
    r = pl.program_id(0)
    slot = _run_prefetch(scal_ref, r, (gw_hbm, uw_hbm), (g0, u0), (g1, u1),
                         (sg0, su0), (sg1, su1))
    xb = xs_ref[...]

    def compute(gb, ub):
        g = lax.dot_general(xb, gb[...], (((1,), (1,)), ((), ())),
                            preferred_element_type=jnp.float32)
        u = lax.dot_general(xb, ub[...], (((1,), (1,)), ((), ())),
                            preferred_element_type=jnp.float32)
        return ((g * jax.nn.sigmoid(g)) * u * wv_ref[...]).astype(h_ref.dtype)

    @pl.when(slot == 0)
    def _():
        h_ref[...] = compute(g0, u0)

    @pl.when(slot == 1)
    def _():
        h_ref[...] = compute(g1, u1)


def _k1(xs, gw, uw, wv, scal):
    n_pad = xs.shape[0]
    n_r = n_pad // T
    gs = pltpu.PrefetchScalarGridSpec(
        num_scalar_prefetch=1,
        grid=(n_r,),
        in_specs=[pl.BlockSpec((T, H), lambda r, s: (r, 0)),
                  pl.BlockSpec(memory_space=pltpu.HBM),
                  pl.BlockSpec(memory_space=pltpu.HBM),
                  pl.BlockSpec((T, 1), lambda r, s: (r, 0))],
        out_specs=pl.BlockSpec((T, I), lambda r, s: (r, 0)),
        scratch_shapes=[pltpu.VMEM((I, H), jnp.float32),
                        pltpu.VMEM((I, H), jnp.float32),
                        pltpu.VMEM((I, H), jnp.float32),
                        pltpu.VMEM((I, H), jnp.float32),
                        pltpu.SemaphoreType.DMA,
                        pltpu.SemaphoreType.DMA,
                        pltpu.SemaphoreType.DMA,
                        pltpu.SemaphoreType.DMA],
    )
    return pl.pallas_call(
        _k1_body, grid_spec=gs,
        out_shape=jax.ShapeDtypeStruct((n_pad, I), jnp.bfloat16),
    )(scal, xs, gw, uw, wv)


def _k2_body(scal_ref, h_ref, dw_hbm, o_ref, d0, d1, sd0, sd1):
    r = pl.program_id(0)
    slot = _run_prefetch(scal_ref, r, (dw_hbm,), (d0,), (d1,), (sd0,), (sd1,))
    hb = h_ref[...].astype(jnp.float32)

    @pl.when(slot == 0)
    def _():
        o_ref[...] = lax.dot_general(hb, d0[...], (((1,), (1,)), ((), ())),
                                     preferred_element_type=jnp.float32)

    @pl.when(slot == 1)
    def _():
        o_ref[...] = lax.dot_general(hb, d1[...], (((1,), (1,)), ((), ())),
                                     preferred_element_type=jnp.float32)


def _k2(h, dw, scal):
    n_pad = h.shape[0]
    n_r = n_pad // T
    gs = pltpu.PrefetchScalarGridSpec(
        num_scalar_prefetch=1,
        grid=(n_r,),
        in_specs=[pl.BlockSpec((T, I), lambda r, s: (r, 0)),
                  pl.BlockSpec(memory_space=pltpu.HBM)],
        out_specs=pl.BlockSpec((T, H), lambda r, s: (r, 0)),
        scratch_shapes=[pltpu.VMEM((H, I), jnp.float32),
                        pltpu.VMEM((H, I), jnp.float32),
                        pltpu.SemaphoreType.DMA,
                        pltpu.SemaphoreType.DMA],
    )
    return pl.pallas_call(
        _k2_body, grid_spec=gs,
        out_shape=jax.ShapeDtypeStruct((n_pad, H), jnp.float32),
    )(scal, h, dw)


def _shared_body(x_ref, gw_ref, uw_ref, dw_ref, o_ref):
    xb = x_ref[...]
    g = lax.dot_general(xb, gw_ref[...], (((1,), (1,)), ((), ())),
                        preferred_element_type=jnp.float32)
    u = lax.dot_general(xb, uw_ref[...], (((1,), (1,)), ((), ())),
                        preferred_element_type=jnp.float32)
    h = (g * jax.nn.sigmoid(g)) * u
    o_ref[...] = lax.dot_general(h, dw_ref[...], (((1,), (1,)), ((), ())),
                                 preferred_element_type=jnp.float32)


def _shared(xf, sgw, suw, sdw):
    nt = xf.shape[0]
    return pl.pallas_call(
        _shared_body,
        grid=(nt // T2,),
        in_specs=[pl.BlockSpec((T2, H), lambda r: (r, 0)),
                  pl.BlockSpec((I, H), lambda r: (0, 0)),
                  pl.BlockSpec((I, H), lambda r: (0, 0)),
                  pl.BlockSpec((H, I), lambda r: (0, 0))],
        out_specs=pl.BlockSpec((T2, H), lambda r: (r, 0)),
        out_shape=jax.ShapeDtypeStruct((nt, H), jnp.float32),
    )(xf, sgw, suw, sdw)


def _add_body(a_ref, b_ref, c_ref, o_ref):
    o_ref[...] = a_ref[...] + b_ref[...] + c_ref[...]


def _add3(a, b, c):
    nt = a.shape[0]
    spec = pl.BlockSpec((T2, H), lambda r: (r, 0))
    return pl.pallas_call(
        _add_body,
        grid=(nt // T2,),
        in_specs=[spec, spec, spec],
        out_specs=spec,
        out_shape=jax.ShapeDtypeStruct((nt, H), jnp.float32),
    )(a, b, c)


def kernel(x, gate_w, expert_gate_w, expert_up_w, expert_down_w,
           shared_gate_w, shared_up_w, shared_down_w):
    orig_shape = x.shape
    xf = x.reshape(-1, H)
    nt = xf.shape[0]
    n_pad = nt * TOPK + E * T
    n_r_tiles = n_pad // T

    gwp = jnp.zeros((128, H), jnp.float32).at[:E].set(gate_w)
    idx_full, w_full = _router(xf, gwp)
    ti = idx_full[:, :TOPK]
    tw = w_full[:, :TOPK]

    flat_e = ti.reshape(-1)
    oh = (flat_e[:, None] == jnp.arange(E, dtype=flat_e.dtype)).astype(jnp.int32)
    counts = oh.sum(axis=0)
    padded = ((counts + T - 1) // T) * T
    offs = jnp.concatenate([jnp.zeros((1,), padded.dtype),
                            jnp.cumsum(padded)[:-1]])
    ranks = jnp.sum((jnp.cumsum(oh, axis=0) - oh) * oh, axis=1)
    dest = (jnp.sum(oh * offs[None, :], axis=1) + ranks).astype(jnp.int32)
    tok_ids = jnp.arange(nt * TOPK, dtype=jnp.int32) // TOPK
    pairs = jnp.stack([tok_ids.astype(jnp.float32), tw.reshape(-1)], axis=1)
    slots = jnp.zeros((n_pad, 2), jnp.float32).at[dest].set(pairs)
    src_tok = slots[:, 0].astype(jnp.int32)
    wv = slots[:, 1:2]
    tile_start = jnp.arange(n_r_tiles, dtype=offs.dtype)[:, None] * T
    tile_e = (jnp.sum((tile_start >= offs[None, :]).astype(jnp.int32), axis=1)
              - 1).clip(0, E - 1).astype(jnp.int32)
    scal = _weight_sched(tile_e, padded)

    sh = _shared(xf, shared_gate_w, shared_up_w, shared_down_w)
    xs = _sc_gather(xf, src_tok)
    h = _k1(xs, expert_gate_w, expert_up_w, wv, scal)
    outs = _k2(h, expert_down_w, scal)

    pos = dest.reshape(nt, TOPK)
    comb = _sc_combine(outs, pos[:, 0], pos[:, 1], sh)
    return comb.reshape(orig_shape)
